# Initial kernel scaffold; baseline (speedup 1.0000x reference)
import jax
import jax.numpy as jnp
from jax.experimental import pallas as pl


def kernel(nodes, edges, node_params, edge_params, graph):
    raise NotImplementedError("write your pallas kernel here")


# speedup vs baseline: 3.4369x; 3.4369x over previous
import functools

import jax
import jax.numpy as jnp
from jax import lax
from jax.experimental import pallas as pl
from jax.experimental.pallas import tpu as pltpu
from jax.experimental.pallas import tpu_sc as plsc

NN = 10000
NE = 320000
D = 128
C = 128
NCH = NE // C
NW = 32
JMAX = -(-NCH // NW)
NSUB = 16
ZR = 624
ZTAIL = NN - NSUB * ZR

_mesh = plsc.VectorSubcoreMesh(core_axis_name="c", subcore_axis_name="s")


@functools.partial(
    pl.kernel,
    out_type=jax.ShapeDtypeStruct((2 * NN, D), jnp.float32),
    mesh=_mesh,
    scratch_types=[
        pltpu.VMEM((1, C), jnp.int32),
        pltpu.VMEM((C, D), jnp.float32),
        pltpu.VMEM_SHARED((NN, D), jnp.float32),
    ],
)
def _sc_scatter(edges_hbm, dst3d_hbm, zeros_hbm, out_hbm, idx_v, ed_v, acc_sh):
    c = lax.axis_index("c")
    s = lax.axis_index("s")
    wid = s * 2 + c
    pltpu.sync_copy(zeros_hbm.at[pl.ds(s * ZR, ZR)],
                    acc_sh.at[pl.ds(s * ZR, ZR)])

    @pl.when(s == NSUB - 1)
    def _():
        pltpu.sync_copy(zeros_hbm.at[pl.ds(NSUB * ZR, ZTAIL)],
                        acc_sh.at[pl.ds(NSUB * ZR, ZTAIL)])

    plsc.subcore_barrier()

    def body(j, carry):
        k = wid + NW * j

        @pl.when(k < NCH)
        def _():
            pltpu.sync_copy(dst3d_hbm.at[k], idx_v)
            pltpu.sync_copy(edges_hbm.at[pl.ds(k * C, C)], ed_v)
            pltpu.sync_copy(ed_v, acc_sh.at[idx_v.at[0]], add=True)

        return carry

    lax.fori_loop(0, JMAX, body, 0)
    plsc.subcore_barrier()
    pltpu.sync_copy(acc_sh.at[pl.ds(s * ZR, ZR)],
                    out_hbm.at[pl.ds(c * NN + s * ZR, ZR)])

    @pl.when(s == NSUB - 1)
    def _():
        pltpu.sync_copy(acc_sh.at[pl.ds(NSUB * ZR, ZTAIL)],
                        out_hbm.at[pl.ds(c * NN + NSUB * ZR, ZTAIL)])


@functools.partial(
    pl.kernel,
    out_type=jax.ShapeDtypeStruct((NE, D), jnp.float32),
    mesh=_mesh,
    scratch_types=[
        pltpu.VMEM((1, C), jnp.int32),
        pltpu.VMEM((1, C), jnp.int32),
        pltpu.VMEM((C, D), jnp.float32),
        pltpu.VMEM((C, D), jnp.float32),
        pltpu.SemaphoreType.DMA,
        pltpu.SemaphoreType.DMA,
    ],
)
def _sc_gather(a_hbm, b_hbm, src3d_hbm, dst3d_hbm, out_hbm,
               idxa, idxb, bufa, bufb, sema, semb):
    c = lax.axis_index("c")
    s = lax.axis_index("s")
    wid = s * 2 + c

    def body(j, carry):
        k = wid + NW * j

        @pl.when(k < NCH)
        def _():
            pltpu.sync_copy(src3d_hbm.at[k], idxa)
            pltpu.sync_copy(dst3d_hbm.at[k], idxb)
            cpa = pltpu.async_copy(a_hbm.at[idxa.at[0]], bufa, sema)
            cpb = pltpu.async_copy(b_hbm.at[idxb.at[0]], bufb, semb)
            cpa.wait()
            cpb.wait()

            def addrow(r, cr):
                for u in range(8):
                    plsc.addupdate(bufa.at[r, pl.ds(u * 16, 16)],
                                   bufb[r, pl.ds(u * 16, 16)])
                return cr

            lax.fori_loop(0, C, addrow, 0)
            pltpu.sync_copy(bufa, out_hbm.at[pl.ds(k * C, C)])

        return carry

    lax.fori_loop(0, JMAX, body, 0)


def _ln(x, g, b):
    m = jnp.mean(x, axis=-1, keepdims=True)
    xc = x - m
    v = jnp.mean(xc * xc, axis=-1, keepdims=True)
    return xc * lax.rsqrt(v + 1e-5) * g + b


def _silu(x):
    return x * jax.nn.sigmoid(x)


def _node_body(p_ref, n_ref, w1a, w1b, b1, g1, bb1, w2, b2, g2, bb2, ws, wd,
               nn_ref, a_ref, b_ref):
    msg = p_ref[0:NN, :] + p_ref[NN:2 * NN, :]
    nodes = n_ref[...]
    x = (jnp.dot(nodes, w1a[...], preferred_element_type=jnp.float32)
         + jnp.dot(msg, w1b[...], preferred_element_type=jnp.float32)
         + b1[...])
    x = _silu(_ln(x, g1[...], bb1[...]))
    x = jnp.dot(x, w2[...], preferred_element_type=jnp.float32) + b2[...]
    x = _silu(_ln(x, g2[...], bb2[...]))
    nn = x + nodes
    nn_ref[...] = nn
    a_ref[...] = jnp.dot(nn, ws[...], preferred_element_type=jnp.float32)
    b_ref[...] = jnp.dot(nn, wd[...], preferred_element_type=jnp.float32)


BLK = 2000


def _edge_body(g_ref, e_ref, we, b1, g1, bb1, w2, b2, g2, bb2, out_ref):
    e = e_ref[...]
    h = (g_ref[...]
         + jnp.dot(e, we[...], preferred_element_type=jnp.float32)
         + b1[...])
    h = _silu(_ln(h, g1[...], bb1[...]))
    h = jnp.dot(h, w2[...], preferred_element_type=jnp.float32) + b2[...]
    h = _ln(h, g2[...], bb2[...])
    out_ref[...] = jnp.tanh(h) + e


def _row2d(v):
    return v.reshape(1, D)


def kernel(nodes, edges, node_params, edge_params, graph):
    graph = graph.astype(jnp.int32)
    src3d = graph[0].reshape(NCH, 1, C)
    dst3d = graph[1].reshape(NCH, 1, C)
    zeros = jnp.zeros((NN, D), jnp.float32)

    partials = _sc_scatter(edges, dst3d, zeros)

    np0, np1 = node_params
    ep0, ep1 = edge_params
    w1a, w1b = np0['W'][:D], np0['W'][D:]
    ws, wd, we = ep0['W'][:D], ep0['W'][D:2 * D], ep0['W'][2 * D:]

    full = pl.BlockSpec((D, D), lambda i: (0, 0))
    row = pl.BlockSpec((1, D), lambda i: (0, 0))

    nodes_new, a_arr, b_arr = pl.pallas_call(
        _node_body,
        out_shape=[jax.ShapeDtypeStruct((NN, D), jnp.float32)] * 3,
    )(partials, nodes, w1a, w1b, _row2d(np0['b']), _row2d(np0['g']),
      _row2d(np0['beta']), np1['W'], _row2d(np1['b']), _row2d(np1['g']),
      _row2d(np1['beta']), ws, wd)

    g_arr = _sc_gather(a_arr, b_arr, src3d, dst3d)

    blk = pl.BlockSpec((BLK, D), lambda i: (i, 0))
    edges_new = pl.pallas_call(
        _edge_body,
        grid=(NE // BLK,),
        in_specs=[blk, blk, full, row, row, row, full, row, row, row],
        out_specs=blk,
        out_shape=jax.ShapeDtypeStruct((NE, D), jnp.float32),
    )(g_arr, edges, we, _row2d(ep0['b']), _row2d(ep0['g']),
      _row2d(ep0['beta']), ep1['W'], _row2d(ep1['b']), _row2d(ep1['g']),
      _row2d(ep1['beta']))

    return nodes_new, edges_new


# speedup vs baseline: 4.7398x; 1.3791x over previous
import functools

import jax
import jax.numpy as jnp
import numpy as np
from jax import lax
from jax.experimental import pallas as pl
from jax.experimental.pallas import tpu as pltpu
from jax.experimental.pallas import tpu_sc as plsc

NN = 10000
NE = 320000
D = 128
C = 128
NCH = NE // C
NW = 32
JMAX = -(-NCH // NW)
JPAD = 80
NSUB = 16

_ORDER = np.minimum(
    np.arange(NW)[:, None] + NW * np.arange(JPAD)[None, :], NCH - 1
).reshape(-1)
ZR = 624
ZTAIL = NN - NSUB * ZR

_mesh = plsc.VectorSubcoreMesh(core_axis_name="c", subcore_axis_name="s")


@functools.partial(
    pl.kernel,
    out_type=jax.ShapeDtypeStruct((2 * NN, D), jnp.float32),
    mesh=_mesh,
    scratch_types=[
        pltpu.VMEM((JPAD, C), jnp.int32),
        pltpu.VMEM((C, D), jnp.float32),
        pltpu.VMEM((C, D), jnp.float32),
        pltpu.VMEM_SHARED((NN, D), jnp.float32),
        pltpu.SemaphoreType.DMA,
        pltpu.SemaphoreType.DMA,
    ],
)
def _sc_scatter(edges_hbm, dstord_hbm, zeros_hbm, out_hbm,
                idx_all, ed0, ed1, acc_sh, sem0, sem1):
    c = lax.axis_index("c")
    s = lax.axis_index("s")
    w = s * 2 + c
    ed = [ed0, ed1]
    sem = [sem0, sem1]
    pltpu.sync_copy(zeros_hbm.at[pl.ds(s * ZR, ZR)],
                    acc_sh.at[pl.ds(s * ZR, ZR)])

    @pl.when(s == NSUB - 1)
    def _():
        pltpu.sync_copy(zeros_hbm.at[pl.ds(NSUB * ZR, ZTAIL)],
                        acc_sh.at[pl.ds(NSUB * ZR, ZTAIL)])

    pltpu.sync_copy(dstord_hbm.at[pl.ds(w * JPAD, JPAD)], idx_all)
    plsc.subcore_barrier()

    def issue(j, p):
        k = w + NW * j
        pltpu.async_copy(edges_hbm.at[pl.ds(k * C, C)], ed[p], sem[p])

    def wait(j, p):
        k = w + NW * j
        pltpu.make_async_copy(edges_hbm.at[pl.ds(k * C, C)], ed[p],
                              sem[p]).wait()

    issue(0, 0)

    def outer(t, carry):
        for b in range(2):
            j = 2 * t + b
            p = b
            wait(j, p)

            @pl.when(w + NW * (j + 1) < NCH)
            def _():
                issue(j + 1, 1 - p)

            pltpu.sync_copy(ed[p], acc_sh.at[idx_all.at[j]], add=True)
        return carry

    lax.fori_loop(0, (JMAX - 1) // 2, outer, 0)

    @pl.when(w + NW * (JMAX - 1) < NCH)
    def _():
        wait(JMAX - 1, 0)
        pltpu.sync_copy(ed[0], acc_sh.at[idx_all.at[JMAX - 1]], add=True)

    plsc.subcore_barrier()
    pltpu.sync_copy(acc_sh.at[pl.ds(s * ZR, ZR)],
                    out_hbm.at[pl.ds(c * NN + s * ZR, ZR)])

    @pl.when(s == NSUB - 1)
    def _():
        pltpu.sync_copy(acc_sh.at[pl.ds(NSUB * ZR, ZTAIL)],
                        out_hbm.at[pl.ds(c * NN + NSUB * ZR, ZTAIL)])


@functools.partial(
    pl.kernel,
    out_type=jax.ShapeDtypeStruct((NE, D), jnp.float32),
    mesh=_mesh,
    scratch_types=[
        pltpu.VMEM((JPAD, C), jnp.int32),
        pltpu.VMEM((JPAD, C), jnp.int32),
        pltpu.VMEM((C, D), jnp.float32),
        pltpu.VMEM((C, D), jnp.float32),
        pltpu.VMEM((C, D), jnp.float32),
        pltpu.VMEM((C, D), jnp.float32),
        pltpu.SemaphoreType.DMA,
        pltpu.SemaphoreType.DMA,
        pltpu.SemaphoreType.DMA,
        pltpu.SemaphoreType.DMA,
        pltpu.SemaphoreType.DMA,
        pltpu.SemaphoreType.DMA,
    ],
)
def _sc_gather(a_hbm, b_hbm, srcord_hbm, dstord_hbm, out_hbm,
               idxa_all, idxb_all, bufa0, bufa1, bufb0, bufb1,
               sema0, sema1, semb0, semb1, semo0, semo1):
    c = lax.axis_index("c")
    s = lax.axis_index("s")
    w = s * 2 + c
    bufa = [bufa0, bufa1]
    bufb = [bufb0, bufb1]
    sema = [sema0, sema1]
    semb = [semb0, semb1]
    semo = [semo0, semo1]

    pltpu.sync_copy(srcord_hbm.at[pl.ds(w * JPAD, JPAD)], idxa_all)
    pltpu.sync_copy(dstord_hbm.at[pl.ds(w * JPAD, JPAD)], idxb_all)

    def kof(j):
        return jnp.minimum(w + NW * j, NCH - 1)

    def issue(j, p):
        pltpu.async_copy(a_hbm.at[idxa_all.at[j]], bufa[p], sema[p])
        pltpu.async_copy(b_hbm.at[idxb_all.at[j]], bufb[p], semb[p])

    def wait(j, p):
        pltpu.make_async_copy(a_hbm.at[idxa_all.at[j]], bufa[p],
                              sema[p]).wait()
        pltpu.make_async_copy(b_hbm.at[idxb_all.at[j]], bufb[p],
                              semb[p]).wait()

    def wait_out(j, p):
        pltpu.make_async_copy(
            bufa[p], out_hbm.at[pl.ds(kof(j) * C, C)], semo[p]).wait()

    issue(0, 0)

    def outer(t, carry):
        for b in range(2):
            j = 2 * t + b
            p = b
            wait(j, p)

            @pl.when(j + 1 < JPAD)
            def _():
                @pl.when(j >= 1)
                def _():
                    wait_out(j - 1, 1 - p)

                issue(j + 1, 1 - p)

            def addrow(r, cr):
                for u in range(8):
                    plsc.addupdate(bufa[p].at[r, pl.ds(u * 16, 16)],
                                   bufb[p][r, pl.ds(u * 16, 16)])
                return cr

            lax.fori_loop(0, C, addrow, 0)
            pltpu.async_copy(bufa[p], out_hbm.at[pl.ds(kof(j) * C, C)],
                             semo[p])
        return carry

    lax.fori_loop(0, JPAD // 2, outer, 0)
    wait_out(JPAD - 2, 0)
    wait_out(JPAD - 1, 1)


def _ln(x, g, b):
    m = jnp.mean(x, axis=-1, keepdims=True)
    xc = x - m
    v = jnp.mean(xc * xc, axis=-1, keepdims=True)
    return xc * lax.rsqrt(v + 1e-5) * g + b


def _silu(x):
    return x * jax.nn.sigmoid(x)


def _node_body(p_ref, n_ref, w1a, w1b, b1, g1, bb1, w2, b2, g2, bb2, ws, wd,
               nn_ref, a_ref, b_ref):
    msg = p_ref[0:NN, :] + p_ref[NN:2 * NN, :]
    nodes = n_ref[...]
    x = (jnp.dot(nodes, w1a[...], preferred_element_type=jnp.float32)
         + jnp.dot(msg, w1b[...], preferred_element_type=jnp.float32)
         + b1[...])
    x = _silu(_ln(x, g1[...], bb1[...]))
    x = jnp.dot(x, w2[...], preferred_element_type=jnp.float32) + b2[...]
    x = _silu(_ln(x, g2[...], bb2[...]))
    nn = x + nodes
    nn_ref[...] = nn
    a_ref[...] = jnp.dot(nn, ws[...], preferred_element_type=jnp.float32)
    b_ref[...] = jnp.dot(nn, wd[...], preferred_element_type=jnp.float32)


BLK = 2000


def _edge_body(g_ref, e_ref, we, b1, g1, bb1, w2, b2, g2, bb2, out_ref):
    e = e_ref[...]
    h = (g_ref[...]
         + jnp.dot(e, we[...], preferred_element_type=jnp.float32)
         + b1[...])
    h = _silu(_ln(h, g1[...], bb1[...]))
    h = jnp.dot(h, w2[...], preferred_element_type=jnp.float32) + b2[...]
    h = _ln(h, g2[...], bb2[...])
    out_ref[...] = jnp.tanh(h) + e


def _row2d(v):
    return v.reshape(1, D)


def kernel(nodes, edges, node_params, edge_params, graph):
    graph = graph.astype(jnp.int32)
    order = jnp.asarray(_ORDER, dtype=jnp.int32)
    srcord = jnp.take(graph[0].reshape(NCH, C), order, axis=0)
    dstord = jnp.take(graph[1].reshape(NCH, C), order, axis=0)
    zeros = jnp.zeros((NN, D), jnp.float32)

    partials = _sc_scatter(edges, dstord, zeros)

    np0, np1 = node_params
    ep0, ep1 = edge_params
    w1a, w1b = np0['W'][:D], np0['W'][D:]
    ws, wd, we = ep0['W'][:D], ep0['W'][D:2 * D], ep0['W'][2 * D:]

    full = pl.BlockSpec((D, D), lambda i: (0, 0))
    row = pl.BlockSpec((1, D), lambda i: (0, 0))

    nodes_new, a_arr, b_arr = pl.pallas_call(
        _node_body,
        out_shape=[jax.ShapeDtypeStruct((NN, D), jnp.float32)] * 3,
    )(partials, nodes, w1a, w1b, _row2d(np0['b']), _row2d(np0['g']),
      _row2d(np0['beta']), np1['W'], _row2d(np1['b']), _row2d(np1['g']),
      _row2d(np1['beta']), ws, wd)

    g_arr = _sc_gather(a_arr, b_arr, srcord, dstord)

    blk = pl.BlockSpec((BLK, D), lambda i: (i, 0))
    edges_new = pl.pallas_call(
        _edge_body,
        grid=(NE // BLK,),
        in_specs=[blk, blk, full, row, row, row, full, row, row, row],
        out_specs=blk,
        out_shape=jax.ShapeDtypeStruct((NE, D), jnp.float32),
    )(g_arr, edges, we, _row2d(ep0['b']), _row2d(ep0['g']),
      _row2d(ep0['beta']), ep1['W'], _row2d(ep1['b']), _row2d(ep1['g']),
      _row2d(ep1['beta']))

    return nodes_new, edges_new
